# baseline (speedup 1.0000x reference)
import jax
import jax.numpy as jnp
from jax import lax
from jax.experimental import pallas as pl
from jax.experimental.pallas import tpu as pltpu
from jax.experimental.pallas import tpu_sc as plsc

_B, _H, _W = 128, 256, 256
_N = _H * _W
_K = int(0.75 * _N)
_NW = 32
_RPW = _B // _NW
_NV = _N // 16
_SLOTS = 8


def _key(v):
    u = lax.bitcast_convert_type(v, jnp.int32)
    return u ^ ((u >> 31) & jnp.int32(0x7FFFFFFF))


def _sc_body(imp_hbm, mask_hbm, rs_hbm, data, hist, tot, rs_v, state):
    wid = lax.axis_index("s") * 2 + lax.axis_index("c")
    lanes = lax.iota(jnp.int32, 16)
    ones = jnp.ones((16,), jnp.int32)
    zeros16 = jnp.zeros((16,), jnp.int32)

    @plsc.parallel_loop(0, _SLOTS * 256, unroll=8)
    def _zero(i):
        hist[pl.ds(i * 16, 16)] = zeros16

    def row_body(j, rs_vec):
        row = wid * _RPW + j
        pltpu.sync_copy(imp_hbm.at[row], data)
        state[0] = jnp.int32(0)
        state[1] = jnp.int32(_K)
        state[2] = jnp.int32(0)
        state[3] = jnp.int32(0)
        state[4] = jnp.int32(0)
        state[5] = jnp.int32(0)

        def level_block(lev):
            @pl.when(state[0] == 0)
            def _():
                r = state[1]
                pk = state[2]
                pi = state[3]

                @plsc.parallel_loop(0, _NV, unroll=_SLOTS)
                def _scan(i):
                    if lev == 0:
                        ks = _key(data[pl.ds(i * 16, 16)])
                        data[pl.ds(i * 16, 16)] = plsc.bitcast(ks, jnp.float32)
                    else:
                        ks = plsc.bitcast(data[pl.ds(i * 16, 16)], jnp.int32)
                    if lev == 0:
                        d = (ks >> 24) + 128
                        pm = None
                    elif lev == 1:
                        d = (ks >> 16) & 255
                        pm = (ks >> 24) == (pk >> 24)
                    elif lev == 2:
                        d = (ks >> 8) & 255
                        pm = (ks >> 16) == (pk >> 16)
                    elif lev == 3:
                        d = ks & 255
                        pm = (ks >> 8) == (pk >> 8)
                    else:
                        inv = (jnp.int32(_N - 1) - i * 16) - lanes
                        if lev == 4:
                            d = inv >> 8
                            pm = ks == pk
                        else:
                            d = inv & 255
                            pm = (ks == pk) & ((inv >> 8) == (pi >> 8))
                    idx = ((i & (_SLOTS - 1)) << 12) + (d << 4) + lanes
                    if pm is None:
                        plsc.addupdate_scatter(hist, [idx], ones)
                    else:
                        plsc.addupdate_scatter(hist, [idx], ones, mask=pm)

                def sel_a(g, ct):
                    tv = zeros16
                    for m in range(16):
                        base = (g * 16 + m) * 16
                        hv = hist[pl.ds(base, 16)]
                        hist[pl.ds(base, 16)] = zeros16
                        for s in range(1, _SLOTS):
                            hv = hv + hist[pl.ds(s * 4096 + base, 16)]
                            hist[pl.ds(s * 4096 + base, 16)] = zeros16
                        tv = jnp.where(lanes == m, jnp.sum(hv), tv)
                    tot[pl.ds(g * 16, 16)] = tv
                    return jnp.where(lanes == g, jnp.sum(tv), ct)
                chunk_tot = plsc.parallel_loop(0, 16, carry=zeros16)(sel_a)

                rv_c = lax.rev(chunk_tot, (0,))
                cs_c = plsc.cumsum(rv_c)
                fc = jnp.max(plsc.all_reduce_ffs(cs_c >= r))
                cumbef_c = jnp.sum(jnp.where(lanes < fc, rv_c, jnp.int32(0)))
                cstar = jnp.int32(15) - fc
                v = tot[pl.ds(cstar * 16, 16)]
                rv = lax.rev(v, (0,))
                cs2 = plsc.cumsum(rv)
                f2 = jnp.max(plsc.all_reduce_ffs(cs2 >= (r - cumbef_c)))
                bstar = cstar * 16 + jnp.int32(15) - f2
                cumbef = cumbef_c + jnp.sum(
                    jnp.where(lanes < f2, rv, jnp.int32(0)))
                sstar = jnp.sum(jnp.where(lanes == f2, rv, jnp.int32(0)))

                exact = (r - cumbef) == sstar
                if lev < 4:
                    shift = 24 - 8 * lev
                    bs = bstar - 128 if lev == 0 else bstar
                    floor = pk | (bs << shift)
                    state[4] = jnp.where(exact, floor, state[4])
                    state[5] = jnp.where(exact, jnp.int32(0), state[5])
                    state[2] = jnp.where(exact, pk, floor)
                elif lev == 4:
                    floor = bstar << 8
                    state[4] = jnp.where(exact, pk, state[4])
                    state[5] = jnp.where(exact, floor, state[5])
                    state[3] = jnp.where(exact, pi, floor)
                else:
                    state[4] = pk
                    state[5] = pi | bstar
                if lev < 5:
                    state[1] = jnp.where(exact, r, r - cumbef)
                    state[0] = jnp.where(exact, jnp.int32(1), jnp.int32(0))
                else:
                    state[0] = jnp.int32(1)

        for lev in range(6):
            level_block(lev)

        kt = state[4]
        it = state[5]

        @pl.when(it == 0)
        def _fast_emit():
            @plsc.parallel_loop(0, _NV, unroll=8)
            def _em(i):
                ks = plsc.bitcast(data[pl.ds(i * 16, 16)], jnp.int32)
                data[pl.ds(i * 16, 16)] = jnp.where(
                    ks >= kt, jnp.float32(1.0), jnp.float32(0.0))

        @pl.when(it != 0)
        def _tie_emit():
            @plsc.parallel_loop(0, _NV, unroll=8)
            def _em(i):
                ks = plsc.bitcast(data[pl.ds(i * 16, 16)], jnp.int32)
                inv = (jnp.int32(_N - 1) - i * 16) - lanes
                m = (ks > kt) | ((ks == kt) & (inv >= it))
                data[pl.ds(i * 16, 16)] = jnp.where(m, jnp.float32(1.0),
                                                    jnp.float32(0.0))

        pltpu.sync_copy(data, mask_hbm.at[row])
        return jnp.where(lanes == j, jnp.float32(_K), rs_vec)

    rs_vec = lax.fori_loop(0, _RPW, row_body, jnp.zeros((16,), jnp.float32))
    rs_v[...] = rs_vec
    pltpu.sync_copy(rs_v, rs_hbm.at[wid])


def kernel(importance):
    flat = importance.reshape(_B, _N)
    mesh = plsc.VectorSubcoreMesh(core_axis_name="c", subcore_axis_name="s")
    mask, rs = pl.kernel(
        _sc_body,
        mesh=mesh,
        compiler_params=pltpu.CompilerParams(needs_layout_passes=False),
        out_type=[
            jax.ShapeDtypeStruct((_B, _N), jnp.float32),
            jax.ShapeDtypeStruct((_NW, 16), jnp.float32),
        ],
        scratch_types=[
            pltpu.VMEM((_N,), jnp.float32),
            pltpu.VMEM((_SLOTS * 4096,), jnp.int32),
            pltpu.VMEM((256,), jnp.int32),
            pltpu.VMEM((16,), jnp.float32),
            pltpu.SMEM((8,), jnp.int32),
        ],
    )(flat)
    mean = jnp.sum(rs) / jnp.float32(_B * _N)
    return (mask.reshape(_B, 1, _H, _W), mean)


# speedup vs baseline: 1.0839x; 1.0839x over previous
import jax
import jax.numpy as jnp
from jax import lax
from jax.experimental import pallas as pl
from jax.experimental.pallas import tpu as pltpu
from jax.experimental.pallas import tpu_sc as plsc

_B, _H, _W = 128, 256, 256
_N = _H * _W
_K = int(0.75 * _N)
_NW = 32
_RPW = _B // _NW
_NV = _N // 16
_SLOTS = 4
_HALF = _N // 2


def _key(v):
    u = lax.bitcast_convert_type(v, jnp.int32)
    return u ^ ((u >> 31) & jnp.int32(0x7FFFFFFF))


def _sc_body(imp_hbm, mask_hbm, rs_hbm, data, hist, tot, outbuf, rs_v, state,
             sem_in0, sem_in1, sem_out0, sem_out1):
    wid = lax.axis_index("s") * 2 + lax.axis_index("c")
    lanes = lax.iota(jnp.int32, 16)
    ones = jnp.ones((16,), jnp.int32)
    zeros16 = jnp.zeros((16,), jnp.int32)

    def in0(rw):
        return pltpu.make_async_copy(
            imp_hbm.at[pl.ds(rw * _N, _HALF)], data.at[pl.ds(0, _HALF)],
            sem_in0)

    def in1(rw):
        return pltpu.make_async_copy(
            imp_hbm.at[pl.ds(rw * _N + _HALF, _HALF)],
            data.at[pl.ds(_HALF, _HALF)], sem_in1)

    def out0(rw):
        return pltpu.make_async_copy(
            outbuf, mask_hbm.at[pl.ds(rw * _N, _HALF)], sem_out0)

    def out1(rw):
        return pltpu.make_async_copy(
            data.at[pl.ds(_HALF, _HALF)],
            mask_hbm.at[pl.ds(rw * _N + _HALF, _HALF)], sem_out1)

    @plsc.parallel_loop(0, _SLOTS * 256, unroll=8)
    def _zero(i):
        hist[pl.ds(i * 16, 16)] = zeros16

    def row_body(j, rs_vec):
        row = wid * _RPW + j

        @pl.when(j == 0)
        def _prologue():
            in0(row).start()
            in1(row).start()

        @pl.when(j > 0)
        def _steady():
            out1(row - 1).wait()
            in1(row).start()

        state[0] = jnp.int32(0)
        state[1] = jnp.int32(_K)
        state[2] = jnp.int32(0)
        state[3] = jnp.int32(0)
        state[4] = jnp.int32(0)
        state[5] = jnp.int32(0)

        def level_block(lev):
            @pl.when(state[0] == 0)
            def _():
                r = state[1]
                pk = state[2]
                pi = state[3]

                def scan_range(lo, hi):
                    @plsc.parallel_loop(lo, hi, unroll=_SLOTS)
                    def _scan(i):
                        if lev == 0:
                            ks = _key(data[pl.ds(i * 16, 16)])
                            data[pl.ds(i * 16, 16)] = plsc.bitcast(
                                ks, jnp.float32)
                        else:
                            ks = plsc.bitcast(data[pl.ds(i * 16, 16)],
                                              jnp.int32)
                        if lev == 0:
                            d = (ks >> 24) + 128
                            pm = None
                        elif lev == 1:
                            d = (ks >> 16) & 255
                            pm = (ks >> 24) == (pk >> 24)
                        elif lev == 2:
                            d = (ks >> 8) & 255
                            pm = (ks >> 16) == (pk >> 16)
                        elif lev == 3:
                            d = ks & 255
                            pm = (ks >> 8) == (pk >> 8)
                        else:
                            inv = (jnp.int32(_N - 1) - i * 16) - lanes
                            if lev == 4:
                                d = inv >> 8
                                pm = ks == pk
                            else:
                                d = inv & 255
                                pm = (ks == pk) & ((inv >> 8) == (pi >> 8))
                        idx = ((i & (_SLOTS - 1)) << 12) + (d << 4) + lanes
                        if pm is None:
                            plsc.addupdate_scatter(hist, [idx], ones)
                        else:
                            plsc.addupdate_scatter(hist, [idx], ones, mask=pm)

                if lev == 0:
                    in0(row).wait()
                    scan_range(0, _NV // 2)
                    in1(row).wait()
                    scan_range(_NV // 2, _NV)
                else:
                    scan_range(0, _NV)

                def sel_a(g, ct):
                    tv = zeros16
                    for m in range(16):
                        base = (g * 16 + m) * 16
                        hv = hist[pl.ds(base, 16)]
                        hist[pl.ds(base, 16)] = zeros16
                        for s in range(1, _SLOTS):
                            hv = hv + hist[pl.ds(s * 4096 + base, 16)]
                            hist[pl.ds(s * 4096 + base, 16)] = zeros16
                        tv = jnp.where(lanes == m, jnp.sum(hv), tv)
                    tot[pl.ds(g * 16, 16)] = tv
                    return jnp.where(lanes == g, jnp.sum(tv), ct)
                chunk_tot = plsc.parallel_loop(0, 16, carry=zeros16)(sel_a)

                rv_c = lax.rev(chunk_tot, (0,))
                cs_c = plsc.cumsum(rv_c)
                fc = jnp.max(plsc.all_reduce_ffs(cs_c >= r))
                cumbef_c = jnp.sum(jnp.where(lanes < fc, rv_c, jnp.int32(0)))
                cstar = jnp.int32(15) - fc
                v = tot[pl.ds(cstar * 16, 16)]
                rv = lax.rev(v, (0,))
                cs2 = plsc.cumsum(rv)
                f2 = jnp.max(plsc.all_reduce_ffs(cs2 >= (r - cumbef_c)))
                bstar = cstar * 16 + jnp.int32(15) - f2
                cumbef = cumbef_c + jnp.sum(
                    jnp.where(lanes < f2, rv, jnp.int32(0)))
                sstar = jnp.sum(jnp.where(lanes == f2, rv, jnp.int32(0)))

                exact = (r - cumbef) == sstar
                if lev < 4:
                    shift = 24 - 8 * lev
                    bs = bstar - 128 if lev == 0 else bstar
                    floor = pk | (bs << shift)
                    state[4] = jnp.where(exact, floor, state[4])
                    state[5] = jnp.where(exact, jnp.int32(0), state[5])
                    state[2] = jnp.where(exact, pk, floor)
                elif lev == 4:
                    floor = bstar << 8
                    state[4] = jnp.where(exact, pk, state[4])
                    state[5] = jnp.where(exact, floor, state[5])
                    state[3] = jnp.where(exact, pi, floor)
                else:
                    state[4] = pk
                    state[5] = pi | bstar
                if lev < 5:
                    state[1] = jnp.where(exact, r, r - cumbef)
                    state[0] = jnp.where(exact, jnp.int32(1), jnp.int32(0))
                else:
                    state[0] = jnp.int32(1)

        for lev in range(6):
            level_block(lev)

        kt = state[4]
        it = state[5]

        @pl.when(j > 0)
        def _drain_prev_out0():
            out0(row - 1).wait()

        @pl.when(it == 0)
        def _fast_emit0():
            @plsc.parallel_loop(0, _NV // 2, unroll=8)
            def _em(i):
                ks = plsc.bitcast(data[pl.ds(i * 16, 16)], jnp.int32)
                outbuf[pl.ds(i * 16, 16)] = jnp.where(
                    ks >= kt, jnp.float32(1.0), jnp.float32(0.0))

        @pl.when(it != 0)
        def _tie_emit0():
            @plsc.parallel_loop(0, _NV // 2, unroll=8)
            def _em(i):
                ks = plsc.bitcast(data[pl.ds(i * 16, 16)], jnp.int32)
                inv = (jnp.int32(_N - 1) - i * 16) - lanes
                m = (ks > kt) | ((ks == kt) & (inv >= it))
                outbuf[pl.ds(i * 16, 16)] = jnp.where(
                    m, jnp.float32(1.0), jnp.float32(0.0))

        out0(row).start()

        @pl.when(j < _RPW - 1)
        def _prefetch_next0():
            in0(row + 1).start()

        @pl.when(it == 0)
        def _fast_emit1():
            @plsc.parallel_loop(_NV // 2, _NV, unroll=8)
            def _em(i):
                ks = plsc.bitcast(data[pl.ds(i * 16, 16)], jnp.int32)
                data[pl.ds(i * 16, 16)] = jnp.where(
                    ks >= kt, jnp.float32(1.0), jnp.float32(0.0))

        @pl.when(it != 0)
        def _tie_emit1():
            @plsc.parallel_loop(_NV // 2, _NV, unroll=8)
            def _em(i):
                ks = plsc.bitcast(data[pl.ds(i * 16, 16)], jnp.int32)
                inv = (jnp.int32(_N - 1) - i * 16) - lanes
                m = (ks > kt) | ((ks == kt) & (inv >= it))
                data[pl.ds(i * 16, 16)] = jnp.where(
                    m, jnp.float32(1.0), jnp.float32(0.0))

        out1(row).start()

        return jnp.where(lanes == j, jnp.float32(_K), rs_vec)

    rs_vec = lax.fori_loop(0, _RPW, row_body, jnp.zeros((16,), jnp.float32))
    last = wid * _RPW + _RPW - 1
    out0(last).wait()
    out1(last).wait()
    rs_v[...] = rs_vec
    pltpu.sync_copy(rs_v, rs_hbm.at[wid])


def kernel(importance):
    flat = importance.reshape(_B * _N)
    mesh = plsc.VectorSubcoreMesh(core_axis_name="c", subcore_axis_name="s")
    mask, rs = pl.kernel(
        _sc_body,
        mesh=mesh,
        compiler_params=pltpu.CompilerParams(needs_layout_passes=False),
        out_type=[
            jax.ShapeDtypeStruct((_B * _N,), jnp.float32),
            jax.ShapeDtypeStruct((_NW, 16), jnp.float32),
        ],
        scratch_types=[
            pltpu.VMEM((_N,), jnp.float32),
            pltpu.VMEM((_SLOTS * 4096,), jnp.int32),
            pltpu.VMEM((256,), jnp.int32),
            pltpu.VMEM((_HALF,), jnp.float32),
            pltpu.VMEM((16,), jnp.float32),
            pltpu.SMEM((8,), jnp.int32),
            pltpu.SemaphoreType.DMA,
            pltpu.SemaphoreType.DMA,
            pltpu.SemaphoreType.DMA,
            pltpu.SemaphoreType.DMA,
        ],
    )(flat)
    mean = jnp.sum(rs) / jnp.float32(_B * _N)
    return (mask.reshape(_B, 1, _H, _W), mean)
